# Initial kernel scaffold; baseline (speedup 1.0000x reference)
import jax
import jax.numpy as jnp
from jax.experimental import pallas as pl


def kernel(output, conf_thresh):
    raise NotImplementedError("write your pallas kernel here")


# speedup vs baseline: 2.0153x; 2.0153x over previous
import jax
import jax.numpy as jnp
from jax.experimental import pallas as pl
from jax.experimental.pallas import tpu as pltpu

_NB = 16
_NA = 3
_NC = 80
_NH = 76
_NW = 76
_HW = _NH * _NW
_NBA = _NB * _NA
_AW = (1.25, 2.0, 4.125)
_AH = (1.625, 3.75, 2.875)


def _yolo_block(conf_ref, x_ref, o_ref):
    x = x_ref[0]
    t = conf_ref[0]
    pid = pl.program_id(0)
    a = jax.lax.rem(pid, _NA)
    aw = jnp.where(a == 0, _AW[0], jnp.where(a == 1, _AW[1], _AW[2]))
    ah = jnp.where(a == 0, _AH[0], jnp.where(a == 1, _AH[1], _AH[2]))

    hw = jax.lax.broadcasted_iota(jnp.int32, (1, _HW), 1)
    gx = (hw % _NW).astype(jnp.float32)
    gy = (hw // _NW).astype(jnp.float32)

    inv_w = 1.0 / _NW
    inv_h = 1.0 / _NH
    xs = (jax.nn.sigmoid(x[0:1]) + gx) * inv_w
    ys = (jax.nn.sigmoid(x[1:2]) + gy) * inv_h
    ws = jnp.exp(x[2:3]) * (aw * inv_w)
    hs = jnp.exp(x[3:4]) * (ah * inv_h)
    det = jax.nn.sigmoid(x[4:5])

    cls = x[5:5 + _NC]
    m = jnp.max(cls, axis=0, keepdims=True)
    sumexp = jnp.sum(jnp.exp(cls - m), axis=0, keepdims=True)
    cmax = 1.0 / sumexp
    rows = jax.lax.broadcasted_iota(jnp.int32, (_NC, _HW), 0)
    cid = jnp.min(jnp.where(cls == m, rows, _NC), axis=0,
                  keepdims=True).astype(jnp.float32)

    mask = (det > t).astype(jnp.float32)
    out = jnp.concatenate([xs, ys, ws, hs, det, cmax, cid], axis=0)
    o_ref[0] = out * mask


def kernel(output, conf_thresh):
    x = output.reshape(_NBA, 5 + _NC, _HW)
    out = pl.pallas_call(
        _yolo_block,
        grid=(_NBA,),
        in_specs=[
            pl.BlockSpec(memory_space=pltpu.SMEM),
            pl.BlockSpec((1, 5 + _NC, _HW), lambda i: (i, 0, 0)),
        ],
        out_specs=pl.BlockSpec((1, 7, _HW), lambda i: (i, 0, 0)),
        out_shape=jax.ShapeDtypeStruct((_NBA, 7, _HW), jnp.float32),
    )(conf_thresh, x)
    return out.transpose(0, 2, 1).reshape(_NBA * _HW, 7)


# speedup vs baseline: 3.0742x; 1.5255x over previous
import jax
import jax.numpy as jnp
from jax.experimental import pallas as pl
from jax.experimental.pallas import tpu as pltpu

_NB = 16
_NA = 3
_NC = 80
_NH = 76
_NW = 76
_HW = _NH * _NW
_NBA = _NB * _NA
_AW = (1.25, 2.0, 4.125)
_AH = (1.625, 3.75, 2.875)


def _yolo_block(conf_ref, x_ref, o_ref):
    x = x_ref[0]
    t = conf_ref[0]
    a = pl.program_id(1)
    aw = jnp.where(a == 0, _AW[0], jnp.where(a == 1, _AW[1], _AW[2]))
    ah = jnp.where(a == 0, _AH[0], jnp.where(a == 1, _AH[1], _AH[2]))

    gx = jax.lax.broadcasted_iota(jnp.int32, (1, _NH, _NW), 2).astype(jnp.float32)
    gy = jax.lax.broadcasted_iota(jnp.int32, (1, _NH, _NW), 1).astype(jnp.float32)

    inv_w = 1.0 / _NW
    inv_h = 1.0 / _NH
    xs = (jax.nn.sigmoid(x[0:1]) + gx) * inv_w
    ys = (jax.nn.sigmoid(x[1:2]) + gy) * inv_h
    ws = jnp.exp(x[2:3]) * (aw * inv_w)
    hs = jnp.exp(x[3:4]) * (ah * inv_h)
    det = jax.nn.sigmoid(x[4:5])

    cls = x[5:5 + _NC]
    m = jnp.max(cls, axis=0, keepdims=True)
    sumexp = jnp.sum(jnp.exp(cls - m), axis=0, keepdims=True)
    cmax = 1.0 / sumexp
    rows = jax.lax.broadcasted_iota(jnp.int32, (_NC, _NH, _NW), 0)
    cid = jnp.min(jnp.where(cls == m, rows, _NC), axis=0,
                  keepdims=True).astype(jnp.float32)

    mask = (det > t).astype(jnp.float32)
    out = jnp.concatenate([xs, ys, ws, hs, det, cmax, cid], axis=0)
    o_ref[0] = out * mask


def kernel(output, conf_thresh):
    out = pl.pallas_call(
        _yolo_block,
        grid=(_NB, _NA),
        in_specs=[
            pl.BlockSpec(memory_space=pltpu.SMEM),
            pl.BlockSpec((1, 5 + _NC, _NH, _NW), lambda b, a: (b, a, 0, 0)),
        ],
        out_specs=pl.BlockSpec((1, 7, _NH, _NW), lambda b, a: (b * _NA + a, 0, 0, 0)),
        out_shape=jax.ShapeDtypeStruct((_NBA, 7, _NH, _NW), jnp.float32),
    )(conf_thresh, output)
    return out.reshape(_NBA, 7, _HW).transpose(0, 2, 1).reshape(_NBA * _HW, 7)


# speedup vs baseline: 3.1872x; 1.0368x over previous
import jax
import jax.numpy as jnp
from jax.experimental import pallas as pl
from jax.experimental.pallas import tpu as pltpu

_NB = 16
_NA = 3
_NC = 80
_NH = 76
_NW = 76
_HW = _NH * _NW
_NBA = _NB * _NA
_AW = (1.25, 2.0, 4.125)
_AH = (1.625, 3.75, 2.875)


def _yolo_block(conf_ref, x_ref, o_ref):
    t = conf_ref[0]
    a = pl.program_id(1)
    aw = jnp.where(a == 0, _AW[0], jnp.where(a == 1, _AW[1], _AW[2]))
    ah = jnp.where(a == 0, _AH[0], jnp.where(a == 1, _AH[1], _AH[2]))

    gx = jax.lax.broadcasted_iota(jnp.int32, (_NH, _NW), 1).astype(jnp.float32)
    gy = jax.lax.broadcasted_iota(jnp.int32, (_NH, _NW), 0).astype(jnp.float32)

    inv_w = 1.0 / _NW
    inv_h = 1.0 / _NH
    xs = (jax.nn.sigmoid(x_ref[0, 0]) + gx) * inv_w
    ys = (jax.nn.sigmoid(x_ref[0, 1]) + gy) * inv_h
    ws = jnp.exp(x_ref[0, 2]) * (aw * inv_w)
    hs = jnp.exp(x_ref[0, 3]) * (ah * inv_h)
    det = jax.nn.sigmoid(x_ref[0, 4])

    m = x_ref[0, 5]
    idx = jnp.zeros((_NH, _NW), dtype=jnp.float32)
    for i in range(1, _NC):
        c = x_ref[0, 5 + i]
        gt = c > m
        m = jnp.maximum(m, c)
        idx = jnp.where(gt, jnp.float32(i), idx)
    s = jnp.exp(x_ref[0, 5] - m)
    for i in range(1, _NC):
        s = s + jnp.exp(x_ref[0, 5 + i] - m)
    cmax = 1.0 / s

    mask = (det > t).astype(jnp.float32)
    out = jnp.stack([xs, ys, ws, hs, det, cmax, idx], axis=0)
    o_ref[0] = out * mask


def kernel(output, conf_thresh):
    out = pl.pallas_call(
        _yolo_block,
        grid=(_NB, _NA),
        in_specs=[
            pl.BlockSpec(memory_space=pltpu.SMEM),
            pl.BlockSpec((1, 5 + _NC, _NH, _NW), lambda b, a: (b, a, 0, 0)),
        ],
        out_specs=pl.BlockSpec((1, 7, _NH, _NW), lambda b, a: (b * _NA + a, 0, 0, 0)),
        out_shape=jax.ShapeDtypeStruct((_NBA, 7, _NH, _NW), jnp.float32),
    )(conf_thresh, output)
    return out.reshape(_NBA, 7, _HW).transpose(0, 2, 1).reshape(_NBA * _HW, 7)


# speedup vs baseline: 3.4251x; 1.0746x over previous
import jax
import jax.numpy as jnp
from jax.experimental import pallas as pl
from jax.experimental.pallas import tpu as pltpu

_NB = 16
_NA = 3
_NC = 80
_NH = 76
_NW = 76
_HW = _NH * _NW
_NBA = _NB * _NA
_AW = (1.25, 2.0, 4.125)
_AH = (1.625, 3.75, 2.875)


def _yolo_block(conf_ref, x_ref, o_ref):
    t = conf_ref[0]

    gx = jax.lax.broadcasted_iota(jnp.int32, (_NH, _NW), 1).astype(jnp.float32)
    gy = jax.lax.broadcasted_iota(jnp.int32, (_NH, _NW), 0).astype(jnp.float32)

    inv_w = 1.0 / _NW
    inv_h = 1.0 / _NH
    for a in range(_NA):
        c0 = a * (5 + _NC)
        aw = _AW[a]
        ah = _AH[a]
        xs = (jax.nn.sigmoid(x_ref[0, c0 + 0]) + gx) * inv_w
        ys = (jax.nn.sigmoid(x_ref[0, c0 + 1]) + gy) * inv_h
        ws = jnp.exp(x_ref[0, c0 + 2]) * (aw * inv_w)
        hs = jnp.exp(x_ref[0, c0 + 3]) * (ah * inv_h)
        det = jax.nn.sigmoid(x_ref[0, c0 + 4])

        m = x_ref[0, c0 + 5]
        idx = jnp.zeros((_NH, _NW), dtype=jnp.float32)
        for i in range(1, _NC):
            c = x_ref[0, c0 + 5 + i]
            gt = c > m
            m = jnp.maximum(m, c)
            idx = jnp.where(gt, jnp.float32(i), idx)
        s = jnp.exp(x_ref[0, c0 + 5] - m)
        for i in range(1, _NC):
            s = s + jnp.exp(x_ref[0, c0 + 5 + i] - m)
        cmax = 1.0 / s

        mask = (det > t).astype(jnp.float32)
        out = jnp.stack([xs, ys, ws, hs, det, cmax, idx], axis=0)
        o_ref[a] = out * mask


def kernel(output, conf_thresh):
    out = pl.pallas_call(
        _yolo_block,
        grid=(_NB,),
        in_specs=[
            pl.BlockSpec(memory_space=pltpu.SMEM),
            pl.BlockSpec((1, _NA * (5 + _NC), _NH, _NW), lambda b: (b, 0, 0, 0)),
        ],
        out_specs=pl.BlockSpec((_NA, 7, _NH, _NW), lambda b: (b, 0, 0, 0)),
        out_shape=jax.ShapeDtypeStruct((_NBA, 7, _NH, _NW), jnp.float32),
    )(conf_thresh, output)
    return out.reshape(_NBA, 7, _HW).transpose(0, 2, 1).reshape(_NBA * _HW, 7)


# speedup vs baseline: 3.4277x; 1.0008x over previous
import jax
import jax.numpy as jnp
from jax.experimental import pallas as pl
from jax.experimental.pallas import tpu as pltpu

_NB = 16
_NA = 3
_NC = 80
_NH = 76
_NW = 76
_HW = _NH * _NW
_NBA = _NB * _NA
_AW = (1.25, 2.0, 4.125)
_AH = (1.625, 3.75, 2.875)


def _yolo_block(conf_ref, x0_ref, x1_ref, x2_ref, o_ref):
    t = conf_ref[0]

    gx = jax.lax.broadcasted_iota(jnp.int32, (_NH, _NW), 1).astype(jnp.float32)
    gy = jax.lax.broadcasted_iota(jnp.int32, (_NH, _NW), 0).astype(jnp.float32)

    inv_w = 1.0 / _NW
    inv_h = 1.0 / _NH
    for a, x_ref in enumerate((x0_ref, x1_ref, x2_ref)):
        aw = _AW[a]
        ah = _AH[a]
        xs = (jax.nn.sigmoid(x_ref[0, 0]) + gx) * inv_w
        ys = (jax.nn.sigmoid(x_ref[0, 1]) + gy) * inv_h
        ws = jnp.exp(x_ref[0, 2]) * (aw * inv_w)
        hs = jnp.exp(x_ref[0, 3]) * (ah * inv_h)
        det = jax.nn.sigmoid(x_ref[0, 4])

        m = x_ref[0, 5]
        idx = jnp.zeros((_NH, _NW), dtype=jnp.float32)
        for i in range(1, _NC):
            c = x_ref[0, 5 + i]
            gt = c > m
            m = jnp.maximum(m, c)
            idx = jnp.where(gt, jnp.float32(i), idx)
        s = jnp.exp(x_ref[0, 5] - m)
        for i in range(1, _NC):
            s = s + jnp.exp(x_ref[0, 5 + i] - m)
        cmax = 1.0 / s

        mask = (det > t).astype(jnp.float32)
        out = jnp.stack([xs, ys, ws, hs, det, cmax, idx], axis=0)
        o_ref[a] = out * mask


def kernel(output, conf_thresh):
    slab = pl.BlockSpec((1, 5 + _NC, _NH, _NW), lambda b: (b, 0, 0, 0))
    slab1 = pl.BlockSpec((1, 5 + _NC, _NH, _NW), lambda b: (b, 1, 0, 0))
    slab2 = pl.BlockSpec((1, 5 + _NC, _NH, _NW), lambda b: (b, 2, 0, 0))
    out = pl.pallas_call(
        _yolo_block,
        grid=(_NB,),
        in_specs=[pl.BlockSpec(memory_space=pltpu.SMEM), slab, slab1, slab2],
        out_specs=pl.BlockSpec((_NA, 7, _NH, _NW), lambda b: (b, 0, 0, 0)),
        out_shape=jax.ShapeDtypeStruct((_NBA, 7, _NH, _NW), jnp.float32),
    )(conf_thresh, output, output, output)
    return out.reshape(_NBA, 7, _HW).transpose(0, 2, 1).reshape(_NBA * _HW, 7)


# speedup vs baseline: 3.6059x; 1.0520x over previous
import jax
import jax.numpy as jnp
from jax.experimental import pallas as pl
from jax.experimental.pallas import tpu as pltpu

_NB = 16
_NA = 3
_NC = 80
_NH = 76
_NW = 76
_HW = _NH * _NW
_NBA = _NB * _NA
_AW = (1.25, 2.0, 4.125)
_AH = (1.625, 3.75, 2.875)


def _yolo_block(conf_ref, x_ref, o_ref):
    t = conf_ref[0]

    gx = jax.lax.broadcasted_iota(jnp.int32, (_NH, _NW), 1).astype(jnp.float32)
    gy = jax.lax.broadcasted_iota(jnp.int32, (_NH, _NW), 0).astype(jnp.float32)

    inv_w = 1.0 / _NW
    inv_h = 1.0 / _NH
    for a in range(_NA):
        c0 = a * (5 + _NC)
        aw = _AW[a]
        ah = _AH[a]
        xs = (jax.nn.sigmoid(x_ref[0, c0 + 0]) + gx) * inv_w
        ys = (jax.nn.sigmoid(x_ref[0, c0 + 1]) + gy) * inv_h
        ws = jnp.exp(x_ref[0, c0 + 2]) * (aw * inv_w)
        hs = jnp.exp(x_ref[0, c0 + 3]) * (ah * inv_h)
        det = jax.nn.sigmoid(x_ref[0, c0 + 4])

        m = x_ref[0, c0 + 5]
        idx = jnp.zeros((_NH, _NW), dtype=jnp.float32)
        for i in range(1, _NC):
            c = x_ref[0, c0 + 5 + i]
            gt = c > m
            m = jnp.maximum(m, c)
            idx = jnp.where(gt, jnp.float32(i), idx)
        s = jnp.exp(x_ref[0, c0 + 5] - m)
        for i in range(1, _NC):
            s = s + jnp.exp(x_ref[0, c0 + 5 + i] - m)
        cmax = 1.0 / s

        mask = (det > t).astype(jnp.float32)
        out = jnp.stack([xs, ys, ws, hs, det, cmax, idx], axis=0)
        o_ref[a] = out * mask


def _half(output, conf_thresh, half):
    nb = _NB // 2
    out = pl.pallas_call(
        _yolo_block,
        grid=(nb,),
        in_specs=[
            pl.BlockSpec(memory_space=pltpu.SMEM),
            pl.BlockSpec((1, _NA * (5 + _NC), _NH, _NW),
                         lambda b: (b + half * nb, 0, 0, 0)),
        ],
        out_specs=pl.BlockSpec((_NA, 7, _NH, _NW), lambda b: (b, 0, 0, 0)),
        out_shape=jax.ShapeDtypeStruct((nb * _NA, 7, _NH, _NW), jnp.float32),
    )(conf_thresh, output)
    return out.reshape(nb * _NA, 7, _HW).transpose(0, 2, 1).reshape(-1, 7)


def kernel(output, conf_thresh):
    lo = _half(output, conf_thresh, 0)
    hi = _half(output, conf_thresh, 1)
    return jnp.concatenate([lo, hi], axis=0)
